# Initial kernel scaffold; baseline (speedup 1.0000x reference)
import functools

import jax
import jax.numpy as jnp
from jax import lax
from jax.experimental import pallas as pl
from jax.experimental.pallas import tpu as pltpu
from jax.experimental.pallas import tpu_sc as plsc

D = 128
DE = 64
H = 128
NC = 2
NS = 16
NW = NC * NS


def _relu(v):
    return jnp.maximum(v, 0.0)


def _dot(a, b):
    return jax.lax.dot(a, b, preferred_element_type=jnp.float32)


def _build(n, e, interpret=False):
    per_w = e // NW
    cb = min(400, per_w)
    assert per_w % cb == 0 and cb % 8 == 0
    nch = per_w // cb
    sub = 80 if cb % 80 == 0 else cb
    assert cb % sub == 0 and sub <= 128 and sub % 8 == 0
    nsub = cb // sub
    npt = n // NS
    assert npt * NS == n

    mesh = plsc.VectorSubcoreMesh(core_axis_name="c", subcore_axis_name="s")

    @functools.partial(
        pl.kernel,
        out_type=(jax.ShapeDtypeStruct((e, D), jnp.float32),
                  jax.ShapeDtypeStruct((e, D), jnp.float32)),
        mesh=mesh,
        scratch_types=[
            pltpu.VMEM((cb,), jnp.int32),
            pltpu.VMEM((cb,), jnp.int32),
            pltpu.VMEM((cb, D), jnp.float32),
            pltpu.VMEM((cb, D), jnp.float32),
            pltpu.SemaphoreType.DMA,
            pltpu.SemaphoreType.DMA,
        ],
        interpret=interpret,
    )
    def sc_gather2(ta, tb, src, dst, oa, ob, idx_a, idx_b, rows_a, rows_b,
                   sem_a, sem_b):
        wid = lax.axis_index("s") * NC + lax.axis_index("c")
        base = pl.multiple_of(wid * per_w, 8)

        def chunk(ci, carry):
            off = pl.multiple_of(base + ci * cb, 8)
            pltpu.sync_copy(src.at[pl.ds(off, cb)], idx_a)
            pltpu.sync_copy(dst.at[pl.ds(off, cb)], idx_b)
            copies = []
            for j in range(nsub):
                s = j * sub
                copies.append(pltpu.async_copy(
                    ta.at[idx_a.at[pl.ds(s, sub)]], rows_a.at[pl.ds(s, sub)],
                    sem_a))
                copies.append(pltpu.async_copy(
                    tb.at[idx_b.at[pl.ds(s, sub)]], rows_b.at[pl.ds(s, sub)],
                    sem_b))
            for c in copies:
                c.wait()
            pltpu.sync_copy(rows_a, oa.at[pl.ds(off, cb)])
            pltpu.sync_copy(rows_b, ob.at[pl.ds(off, cb)])
            return carry

        lax.fori_loop(0, nch, chunk, 0)

    @functools.partial(
        pl.kernel,
        out_type=jax.ShapeDtypeStruct((NC, n, DE), jnp.float32),
        mesh=mesh,
        scratch_types=[
            pltpu.VMEM((nsub, sub), jnp.int32),
            pltpu.VMEM((cb, DE), jnp.float32),
            pltpu.VMEM_SHARED((n, DE), jnp.float32),
        ],
        interpret=interpret,
    )
    def sc_scatter(vals, dst, zero, out, idx2, rows, acc):
        cid = lax.axis_index("c")
        sid = lax.axis_index("s")
        wid = sid * NC + cid
        base = pl.multiple_of(wid * per_w, 8)
        arow = pl.multiple_of(sid * npt, 8)

        pltpu.sync_copy(zero.at[pl.ds(arow, npt)], acc.at[pl.ds(arow, npt)])
        plsc.subcore_barrier()

        def chunk(ci, carry):
            off = pl.multiple_of(base + ci * cb, 8)
            pltpu.sync_copy(vals.at[pl.ds(off, cb)], rows)
            for j in range(nsub):
                pltpu.sync_copy(dst.at[pl.ds(off + j * sub, sub)], idx2.at[j])
            for j in range(nsub):
                pltpu.sync_copy(rows.at[pl.ds(j * sub, sub)],
                                acc.at[idx2.at[j]], add=True)
            return carry

        lax.fori_loop(0, nch, chunk, 0)
        plsc.subcore_barrier()
        pltpu.sync_copy(acc.at[pl.ds(arow, npt)],
                        out.at[cid, pl.ds(arow, npt)])

    bn = 2000 if n % 2000 == 0 else n
    gn = n // bn
    be = 4000 if e % 4000 == 0 else e
    ge = e // be

    def node_embed_body(x_ref, w_ref, b_ref, wab_ref, nf_ref, a_ref, b2_ref):
        nf = _relu(_dot(x_ref[...], w_ref[...]) + b_ref[...])
        nf_ref[...] = nf
        ab = _dot(nf, wab_ref[...])
        a_ref[...] = ab[:, :D]
        b2_ref[...] = ab[:, D:]

    node_embed = pl.pallas_call(
        node_embed_body,
        grid=(gn,),
        in_specs=[
            pl.BlockSpec((bn, D), lambda i: (i, 0)),
            pl.BlockSpec((D, D), lambda i: (0, 0)),
            pl.BlockSpec((1, D), lambda i: (0, 0)),
            pl.BlockSpec((D, 2 * D), lambda i: (0, 0)),
        ],
        out_specs=[
            pl.BlockSpec((bn, D), lambda i: (i, 0)),
            pl.BlockSpec((bn, D), lambda i: (i, 0)),
            pl.BlockSpec((bn, D), lambda i: (i, 0)),
        ],
        out_shape=[jax.ShapeDtypeStruct((n, D), jnp.float32)] * 3,
        interpret=interpret,
    )

    def edge_embed_body(ea_ref, w_ref, b_ref, ef_ref):
        ef_ref[...] = _relu(_dot(ea_ref[...], w_ref[...]) + b_ref[...])

    edge_embed = pl.pallas_call(
        edge_embed_body,
        grid=(ge,),
        in_specs=[
            pl.BlockSpec((be, 16), lambda i: (i, 0)),
            pl.BlockSpec((16, DE), lambda i: (0, 0)),
            pl.BlockSpec((1, DE), lambda i: (0, 0)),
        ],
        out_specs=pl.BlockSpec((be, DE), lambda i: (i, 0)),
        out_shape=jax.ShapeDtypeStruct((e, DE), jnp.float32),
        interpret=interpret,
    )

    def edge_mlp_body(sa_ref, sb_ref, ef_ref, wc_ref, b1_ref, w2_ref, b2_ref,
                      out_ref):
        h = _relu(sa_ref[...] + sb_ref[...] + _dot(ef_ref[...], wc_ref[...])
                  + b1_ref[...])
        out_ref[...] = _relu(_dot(h, w2_ref[...]) + b2_ref[...])

    edge_mlp = pl.pallas_call(
        edge_mlp_body,
        grid=(ge,),
        in_specs=[
            pl.BlockSpec((be, D), lambda i: (i, 0)),
            pl.BlockSpec((be, D), lambda i: (i, 0)),
            pl.BlockSpec((be, DE), lambda i: (i, 0)),
            pl.BlockSpec((DE, H), lambda i: (0, 0)),
            pl.BlockSpec((1, H), lambda i: (0, 0)),
            pl.BlockSpec((H, DE), lambda i: (0, 0)),
            pl.BlockSpec((1, DE), lambda i: (0, 0)),
        ],
        out_specs=pl.BlockSpec((be, DE), lambda i: (i, 0)),
        out_shape=jax.ShapeDtypeStruct((e, DE), jnp.float32),
        interpret=interpret,
    )

    def node_update_body(nf_ref, g0_ref, g1_ref, wa_ref, wb_ref, b_ref,
                         wab_ref, nf2_ref, a_ref, b2_ref):
        agg = g0_ref[...] + g1_ref[...]
        nf2 = _relu(_dot(nf_ref[...], wa_ref[...]) + _dot(agg, wb_ref[...])
                    + b_ref[...])
        nf2_ref[...] = nf2
        ab = _dot(nf2, wab_ref[...])
        a_ref[...] = ab[:, :D]
        b2_ref[...] = ab[:, D:]

    node_update = pl.pallas_call(
        node_update_body,
        grid=(gn,),
        in_specs=[
            pl.BlockSpec((bn, D), lambda i: (i, 0)),
            pl.BlockSpec((bn, DE), lambda i: (i, 0)),
            pl.BlockSpec((bn, DE), lambda i: (i, 0)),
            pl.BlockSpec((D, D), lambda i: (0, 0)),
            pl.BlockSpec((DE, D), lambda i: (0, 0)),
            pl.BlockSpec((1, D), lambda i: (0, 0)),
            pl.BlockSpec((D, 2 * D), lambda i: (0, 0)),
        ],
        out_specs=[
            pl.BlockSpec((bn, D), lambda i: (i, 0)),
            pl.BlockSpec((bn, D), lambda i: (i, 0)),
            pl.BlockSpec((bn, D), lambda i: (i, 0)),
        ],
        out_shape=[jax.ShapeDtypeStruct((n, D), jnp.float32)] * 3,
        interpret=interpret,
    )

    def node_final_body(nf_ref, g0_ref, g1_ref, wa_ref, wb_ref, b_ref,
                        wt1_ref, bt1_ref, wt2_ref, bt2_ref, p_ref):
        agg = g0_ref[...] + g1_ref[...]
        nf2 = _relu(_dot(nf_ref[...], wa_ref[...]) + _dot(agg, wb_ref[...])
                    + b_ref[...])
        t = _relu(_dot(nf2, wt1_ref[...]) + bt1_ref[...])
        p_ref[...] = _dot(t, wt2_ref[...]) + bt2_ref[...]

    node_final = pl.pallas_call(
        node_final_body,
        grid=(gn,),
        in_specs=[
            pl.BlockSpec((bn, D), lambda i: (i, 0)),
            pl.BlockSpec((bn, DE), lambda i: (i, 0)),
            pl.BlockSpec((bn, DE), lambda i: (i, 0)),
            pl.BlockSpec((D, D), lambda i: (0, 0)),
            pl.BlockSpec((DE, D), lambda i: (0, 0)),
            pl.BlockSpec((1, D), lambda i: (0, 0)),
            pl.BlockSpec((D, DE), lambda i: (0, 0)),
            pl.BlockSpec((1, DE), lambda i: (0, 0)),
            pl.BlockSpec((DE, 1), lambda i: (0, 0)),
            pl.BlockSpec((1, 1), lambda i: (0, 0)),
        ],
        out_specs=pl.BlockSpec((bn, 1), lambda i: (i, 0)),
        out_shape=jax.ShapeDtypeStruct((n, 1), jnp.float32),
        interpret=interpret,
    )

    def run(x, edge_attr, edge_index, Wn0, bn0, We0, be0, Wm1, bm1, Wm2, bm2,
            Wu, bu, Wt1, bt1, Wt2, bt2):
        src = edge_index[0].astype(jnp.int32)
        dst = edge_index[1].astype(jnp.int32)
        WmA = Wm1[:D]
        WmB = Wm1[D:2 * D]
        WmC = Wm1[2 * D:]
        WAB = jnp.concatenate([WmA, WmB], axis=1)
        WuA = Wu[:D]
        WuB = Wu[D:]
        zero = jnp.zeros((n, DE), jnp.float32)
        b1r = bm1.reshape(1, -1)
        b2r = bm2.reshape(1, -1)

        nf, A, B = node_embed(x, Wn0, bn0.reshape(1, -1), WAB)
        ef = edge_embed(edge_attr, We0, be0.reshape(1, -1))

        sa, sb = sc_gather2(A, B, src, dst)
        ef = edge_mlp(sa, sb, ef, WmC, b1r, Wm2, b2r)
        agg = sc_scatter(ef, dst, zero)
        nf, A, B = node_update(nf, agg[0], agg[1], WuA, WuB,
                               bu.reshape(1, -1), WAB)
        sa, sb = sc_gather2(A, B, src, dst)
        ef = edge_mlp(sa, sb, ef, WmC, b1r, Wm2, b2r)
        agg = sc_scatter(ef, dst, zero)
        preds = node_final(nf, agg[0], agg[1], WuA, WuB, bu.reshape(1, -1),
                           Wt1, bt1.reshape(1, -1), Wt2,
                           bt2.reshape(1, -1))
        return preds.squeeze(-1)

    return run


kernel = _build(10000, 320000)


# speedup vs baseline: 2.8280x; 2.8280x over previous
import functools

import jax
import jax.numpy as jnp
from jax import lax
from jax.experimental import pallas as pl
from jax.experimental.pallas import tpu as pltpu
from jax.experimental.pallas import tpu_sc as plsc

D = 128
DE = 64
H = 128
NC = 2
NS = 16
NW = NC * NS


def _relu(v):
    return jnp.maximum(v, 0.0)


def _dot(a, b):
    return jax.lax.dot(a, b, preferred_element_type=jnp.float32)


def _build(n, e, interpret=False):
    per_w = e // NW
    cb = min(400, per_w)
    assert per_w % cb == 0 and cb % 8 == 0
    nch = per_w // cb
    sub = 80 if cb % 80 == 0 else cb
    assert cb % sub == 0 and sub <= 128 and sub % 8 == 0
    nsub = cb // sub
    npt = -(-n // (NS * 8)) * 8
    n_pad = npt * NS

    mesh = plsc.VectorSubcoreMesh(core_axis_name="c", subcore_axis_name="s",
                                  num_cores=NC, num_subcores=NS)

    @functools.partial(
        pl.kernel,
        out_type=(jax.ShapeDtypeStruct((e, D), jnp.float32),
                  jax.ShapeDtypeStruct((e, D), jnp.float32)),
        mesh=mesh,
        scratch_types=(
            [pltpu.VMEM((sub,), jnp.int32)] * (2 * nsub) + [
                pltpu.VMEM((cb, D), jnp.float32),
                pltpu.VMEM((cb, D), jnp.float32),
                pltpu.SemaphoreType.DMA,
                pltpu.SemaphoreType.DMA,
            ]
        ),
        interpret=interpret,
    )
    def sc_gather2(ta, tb, src, dst, oa, ob, *scr):
        idx_as = scr[:nsub]
        idx_bs = scr[nsub:2 * nsub]
        rows_a, rows_b, sem_a, sem_b = scr[2 * nsub:]
        wid = lax.axis_index("s") * NC + lax.axis_index("c")
        base = pl.multiple_of(wid * per_w, 8)

        def chunk(ci, carry):
            off = pl.multiple_of(base + ci * cb, 8)
            for j in range(nsub):
                pltpu.sync_copy(src.at[pl.ds(off + j * sub, sub)], idx_as[j])
                pltpu.sync_copy(dst.at[pl.ds(off + j * sub, sub)], idx_bs[j])
            copies = []
            for j in range(nsub):
                s = j * sub
                copies.append(pltpu.async_copy(
                    ta.at[idx_as[j]], rows_a.at[pl.ds(s, sub)], sem_a))
                copies.append(pltpu.async_copy(
                    tb.at[idx_bs[j]], rows_b.at[pl.ds(s, sub)], sem_b))
            for c in copies:
                c.wait()
            pltpu.sync_copy(rows_a, oa.at[pl.ds(off, cb)])
            pltpu.sync_copy(rows_b, ob.at[pl.ds(off, cb)])
            return carry

        lax.fori_loop(0, nch, chunk, 0)

    cbs = 200 if per_w % 200 == 0 else per_w
    assert per_w % cbs == 0 and cbs % 8 == 0
    nchs = per_w // cbs
    subs = 40 if cbs % 40 == 0 else cbs
    assert cbs % subs == 0 and subs <= 128 and subs % 8 == 0
    nsubs = cbs // subs
    pieces = [(o, min(cbs, npt - o)) for o in range(0, npt, cbs)]

    @functools.partial(
        pl.kernel,
        out_type=jax.ShapeDtypeStruct((NC * n_pad, D), jnp.float32),
        mesh=mesh,
        scratch_types=(
            [pltpu.VMEM((subs,), jnp.int32)] * nsubs + [
                pltpu.VMEM((cbs, D), jnp.float32),
                pltpu.VMEM_SHARED((n_pad, D), jnp.float32),
            ]
        ),
        interpret=interpret,
    )
    def sc_scatter(vals, dst, zero, out, *scr):
        idxs = scr[:nsubs]
        rows, acc = scr[nsubs:]
        cid = lax.axis_index("c")
        sid = lax.axis_index("s")
        wid = sid * NC + cid
        base = pl.multiple_of(wid * per_w, 8)
        arow = pl.multiple_of(sid * npt, 8)

        pltpu.sync_copy(zero.at[pl.ds(0, cbs)], rows)
        for o, sz in pieces:
            pltpu.sync_copy(rows.at[pl.ds(0, sz)],
                            acc.at[pl.ds(arow + o, sz)])
        plsc.subcore_barrier()

        def chunk(ci, carry):
            off = pl.multiple_of(base + ci * cbs, 8)
            pltpu.sync_copy(vals.at[pl.ds(off, cbs)], rows)
            for j in range(nsubs):
                pltpu.sync_copy(dst.at[pl.ds(off + j * subs, subs)], idxs[j])
            for j in range(nsubs):
                pltpu.sync_copy(rows.at[pl.ds(j * subs, subs)],
                                acc.at[idxs[j]], add=True)
            return carry

        lax.fori_loop(0, nchs, chunk, 0)
        plsc.subcore_barrier()
        obase = pl.multiple_of(cid * n_pad + arow, 8)
        for o, sz in pieces:
            pltpu.sync_copy(acc.at[pl.ds(arow + o, sz)],
                            rows.at[pl.ds(0, sz)])
            pltpu.sync_copy(rows.at[pl.ds(0, sz)],
                            out.at[pl.ds(obase + o, sz)])

    bn = 2000 if n % 2000 == 0 else n
    gn = n // bn
    be = 4000 if e % 4000 == 0 else e
    ge = e // be

    def node_embed_body(x_ref, w_ref, b_ref, wab_ref, nf_ref, a_ref, b2_ref):
        nf = _relu(_dot(x_ref[...], w_ref[...]) + b_ref[...])
        nf_ref[...] = nf
        ab = _dot(nf, wab_ref[...])
        a_ref[...] = ab[:, :D]
        b2_ref[...] = ab[:, D:]

    node_embed = pl.pallas_call(
        node_embed_body,
        grid=(gn,),
        in_specs=[
            pl.BlockSpec((bn, D), lambda i: (i, 0)),
            pl.BlockSpec((D, D), lambda i: (0, 0)),
            pl.BlockSpec((1, D), lambda i: (0, 0)),
            pl.BlockSpec((D, 2 * D), lambda i: (0, 0)),
        ],
        out_specs=[
            pl.BlockSpec((bn, D), lambda i: (i, 0)),
            pl.BlockSpec((bn, D), lambda i: (i, 0)),
            pl.BlockSpec((bn, D), lambda i: (i, 0)),
        ],
        out_shape=[jax.ShapeDtypeStruct((n, D), jnp.float32)] * 3,
        interpret=interpret,
    )

    def edge_embed_body(ea_ref, w_ref, b_ref, ef_ref):
        ef = _relu(_dot(ea_ref[...], w_ref[...]) + b_ref[...])
        ef_ref[...] = jnp.concatenate(
            [ef, jnp.zeros((ef.shape[0], D - DE), jnp.float32)], axis=1)

    edge_embed = pl.pallas_call(
        edge_embed_body,
        grid=(ge,),
        in_specs=[
            pl.BlockSpec((be, 16), lambda i: (i, 0)),
            pl.BlockSpec((16, DE), lambda i: (0, 0)),
            pl.BlockSpec((1, DE), lambda i: (0, 0)),
        ],
        out_specs=pl.BlockSpec((be, D), lambda i: (i, 0)),
        out_shape=jax.ShapeDtypeStruct((e, D), jnp.float32),
        interpret=interpret,
    )

    def edge_mlp_body(sa_ref, sb_ref, ef_ref, wc_ref, b1_ref, w2_ref, b2_ref,
                      out_ref):
        h = _relu(sa_ref[...] + sb_ref[...]
                  + _dot(ef_ref[:, :DE], wc_ref[...]) + b1_ref[...])
        ef2 = _relu(_dot(h, w2_ref[...]) + b2_ref[...])
        out_ref[...] = jnp.concatenate(
            [ef2, jnp.zeros((ef2.shape[0], D - DE), jnp.float32)], axis=1)

    edge_mlp = pl.pallas_call(
        edge_mlp_body,
        grid=(ge,),
        in_specs=[
            pl.BlockSpec((be, D), lambda i: (i, 0)),
            pl.BlockSpec((be, D), lambda i: (i, 0)),
            pl.BlockSpec((be, D), lambda i: (i, 0)),
            pl.BlockSpec((DE, H), lambda i: (0, 0)),
            pl.BlockSpec((1, H), lambda i: (0, 0)),
            pl.BlockSpec((H, DE), lambda i: (0, 0)),
            pl.BlockSpec((1, DE), lambda i: (0, 0)),
        ],
        out_specs=pl.BlockSpec((be, D), lambda i: (i, 0)),
        out_shape=jax.ShapeDtypeStruct((e, D), jnp.float32),
        interpret=interpret,
    )

    def node_update_body(nf_ref, g0_ref, g1_ref, wa_ref, wb_ref, b_ref,
                         wab_ref, nf2_ref, a_ref, b2_ref):
        agg = g0_ref[:, :DE] + g1_ref[:, :DE]
        nf2 = _relu(_dot(nf_ref[...], wa_ref[...]) + _dot(agg, wb_ref[...])
                    + b_ref[...])
        nf2_ref[...] = nf2
        ab = _dot(nf2, wab_ref[...])
        a_ref[...] = ab[:, :D]
        b2_ref[...] = ab[:, D:]

    node_update = pl.pallas_call(
        node_update_body,
        grid=(gn,),
        in_specs=[
            pl.BlockSpec((bn, D), lambda i: (i, 0)),
            pl.BlockSpec((bn, D), lambda i: (i, 0)),
            pl.BlockSpec((bn, D), lambda i: (i, 0)),
            pl.BlockSpec((D, D), lambda i: (0, 0)),
            pl.BlockSpec((DE, D), lambda i: (0, 0)),
            pl.BlockSpec((1, D), lambda i: (0, 0)),
            pl.BlockSpec((D, 2 * D), lambda i: (0, 0)),
        ],
        out_specs=[
            pl.BlockSpec((bn, D), lambda i: (i, 0)),
            pl.BlockSpec((bn, D), lambda i: (i, 0)),
            pl.BlockSpec((bn, D), lambda i: (i, 0)),
        ],
        out_shape=[jax.ShapeDtypeStruct((n, D), jnp.float32)] * 3,
        interpret=interpret,
    )

    def node_final_body(nf_ref, g0_ref, g1_ref, wa_ref, wb_ref, b_ref,
                        wt1_ref, bt1_ref, wt2_ref, bt2_ref, p_ref):
        agg = g0_ref[:, :DE] + g1_ref[:, :DE]
        nf2 = _relu(_dot(nf_ref[...], wa_ref[...]) + _dot(agg, wb_ref[...])
                    + b_ref[...])
        t = _relu(_dot(nf2, wt1_ref[...]) + bt1_ref[...])
        p_ref[...] = _dot(t, wt2_ref[...]) + bt2_ref[...]

    node_final = pl.pallas_call(
        node_final_body,
        grid=(gn,),
        in_specs=[
            pl.BlockSpec((bn, D), lambda i: (i, 0)),
            pl.BlockSpec((bn, D), lambda i: (i, 0)),
            pl.BlockSpec((bn, D), lambda i: (i, 0)),
            pl.BlockSpec((D, D), lambda i: (0, 0)),
            pl.BlockSpec((DE, D), lambda i: (0, 0)),
            pl.BlockSpec((1, D), lambda i: (0, 0)),
            pl.BlockSpec((D, DE), lambda i: (0, 0)),
            pl.BlockSpec((1, DE), lambda i: (0, 0)),
            pl.BlockSpec((DE, 1), lambda i: (0, 0)),
            pl.BlockSpec((1, 1), lambda i: (0, 0)),
        ],
        out_specs=pl.BlockSpec((bn, 1), lambda i: (i, 0)),
        out_shape=jax.ShapeDtypeStruct((n, 1), jnp.float32),
        interpret=interpret,
    )

    def run(x, edge_attr, edge_index, Wn0, bn0, We0, be0, Wm1, bm1, Wm2, bm2,
            Wu, bu, Wt1, bt1, Wt2, bt2):
        src = edge_index[0].astype(jnp.int32)
        dst = edge_index[1].astype(jnp.int32)
        WmA = Wm1[:D]
        WmB = Wm1[D:2 * D]
        WmC = Wm1[2 * D:]
        WAB = jnp.concatenate([WmA, WmB], axis=1)
        WuA = Wu[:D]
        WuB = Wu[D:]
        zero = jnp.zeros((n_pad, D), jnp.float32)
        b1r = bm1.reshape(1, -1)
        b2r = bm2.reshape(1, -1)

        nf, A, B = node_embed(x, Wn0, bn0.reshape(1, -1), WAB)
        ef = edge_embed(edge_attr, We0, be0.reshape(1, -1))

        sa, sb = sc_gather2(A, B, src, dst)
        ef = edge_mlp(sa, sb, ef, WmC, b1r, Wm2, b2r)
        agg = sc_scatter(ef, dst, zero)
        nf, A, B = node_update(nf, agg[:n], agg[n_pad:n_pad + n], WuA, WuB,
                               bu.reshape(1, -1), WAB)
        sa, sb = sc_gather2(A, B, src, dst)
        ef = edge_mlp(sa, sb, ef, WmC, b1r, Wm2, b2r)
        agg = sc_scatter(ef, dst, zero)
        preds = node_final(nf, agg[:n], agg[n_pad:n_pad + n], WuA, WuB,
                           bu.reshape(1, -1),
                           Wt1, bt1.reshape(1, -1), Wt2,
                           bt2.reshape(1, -1))
        return preds.squeeze(-1)

    return run


@functools.cache
def _pipeline():
    return _build(10000, 320000)


def kernel(x, edge_attr, edge_index, Wn0, bn0, We0, be0, Wm1, bm1, Wm2, bm2,
           Wu, bu, Wt1, bt1, Wt2, bt2):
    return _pipeline()(x, edge_attr, edge_index, Wn0, bn0, We0, be0,
                       Wm1, bm1, Wm2, bm2, Wu, bu, Wt1, bt1, Wt2, bt2)


# speedup vs baseline: 3.6440x; 1.2885x over previous
import functools

import jax
import jax.numpy as jnp
from jax import lax
from jax.experimental import pallas as pl
from jax.experimental.pallas import tpu as pltpu
from jax.experimental.pallas import tpu_sc as plsc

D = 128
DE = 64
H = 128
NC = 2
NS = 16
NW = NC * NS


def _relu(v):
    return jnp.maximum(v, 0.0)


def _dot(a, b):
    return jax.lax.dot(a, b, preferred_element_type=jnp.float32)


def _build(n, e, interpret=False):
    per_w = e // NW
    cb = min(400, per_w)
    assert per_w % cb == 0 and cb % 8 == 0
    nch = per_w // cb
    sub = 80 if cb % 80 == 0 else cb
    assert cb % sub == 0 and sub <= 128 and sub % 8 == 0
    nsub = cb // sub
    npt = -(-n // (NS * 8)) * 8
    n_pad = npt * NS

    mesh = plsc.VectorSubcoreMesh(core_axis_name="c", subcore_axis_name="s",
                                  num_cores=NC, num_subcores=NS)

    gcb = 200 if per_w % 400 == 0 else per_w
    assert per_w % gcb == 0 and gcb % 8 == 0
    gnch = per_w // gcb
    assert gnch % 2 == 0 and gnch >= 4
    gsub = 40 if gcb % 40 == 0 else gcb
    assert gcb % gsub == 0 and gsub <= 128 and gsub % 8 == 0
    gnsub = gcb // gsub

    @functools.partial(
        pl.kernel,
        out_type=(jax.ShapeDtypeStruct((e, D), jnp.float32),
                  jax.ShapeDtypeStruct((e, D), jnp.float32)),
        mesh=mesh,
        scratch_types=(
            [pltpu.VMEM((gcb,), jnp.int32)] * 4 +
            [pltpu.VMEM((gcb, D), jnp.float32)] * 4 +
            [pltpu.SemaphoreType.DMA] * 4
        ),
        interpret=interpret,
    )
    def sc_gather2(ta, tb, src, dst, oa, ob, *scr):
        idx_a = scr[0:2]
        idx_b = scr[2:4]
        ra = scr[4:6]
        rb = scr[6:8]
        gsem = scr[8:10]
        wsem = scr[10:12]
        wid = lax.axis_index("s") * NC + lax.axis_index("c")
        base = pl.multiple_of(wid * per_w, 8)

        def off_of(c):
            return pl.multiple_of(base + c * gcb, 8)

        def load_idx(b, c):
            off = off_of(c)
            pltpu.sync_copy(src.at[pl.ds(off, gcb)], idx_a[b])
            pltpu.sync_copy(dst.at[pl.ds(off, gcb)], idx_b[b])

        def g_copies(b):
            res = []
            for j in range(gnsub):
                s = j * gsub
                res.append(pltpu.make_async_copy(
                    ta.at[idx_a[b].at[pl.ds(s, gsub)]],
                    ra[b].at[pl.ds(s, gsub)], gsem[b]))
                res.append(pltpu.make_async_copy(
                    tb.at[idx_b[b].at[pl.ds(s, gsub)]],
                    rb[b].at[pl.ds(s, gsub)], gsem[b]))
            return res

        def start_g(b):
            for c in g_copies(b):
                c.start()

        def wait_g(b):
            for c in g_copies(b):
                c.wait()

        def w_copies(b, c):
            off = off_of(c)
            return [pltpu.make_async_copy(ra[b], oa.at[pl.ds(off, gcb)],
                                          wsem[b]),
                    pltpu.make_async_copy(rb[b], ob.at[pl.ds(off, gcb)],
                                          wsem[b])]

        def start_wb(b, c):
            for cp in w_copies(b, c):
                cp.start()

        def wait_wb(b, c):
            for cp in w_copies(b, c):
                cp.wait()

        load_idx(0, 0)
        start_g(0)
        wait_g(0)
        start_wb(0, 0)
        load_idx(1, 1)
        start_g(1)

        def step(b, c):
            wait_g(b)
            start_wb(b, c)
            wait_wb(1 - b, c - 1)
            load_idx(1 - b, c + 1)
            start_g(1 - b)

        def body(k, carry):
            step(1, 2 * k + 1)
            step(0, 2 * k + 2)
            return carry

        lax.fori_loop(0, gnch // 2 - 1, body, 0)
        wait_g(1)
        start_wb(1, gnch - 1)
        wait_wb(0, gnch - 2)
        wait_wb(1, gnch - 1)

    cbs = 80 if per_w % 80 == 0 else per_w
    assert per_w % cbs == 0 and cbs % 8 == 0 and cbs <= 128
    nchs = per_w // cbs
    assert nchs % 2 == 1 and nchs >= 5
    pieces = [(o, min(cbs, npt - o)) for o in range(0, npt, cbs)]

    @functools.partial(
        pl.kernel,
        out_type=jax.ShapeDtypeStruct((NC * n_pad, D), jnp.float32),
        mesh=mesh,
        scratch_types=(
            [pltpu.VMEM((cbs,), jnp.int32)] * 2 +
            [pltpu.VMEM((cbs, D), jnp.float32)] * 2 +
            [pltpu.SemaphoreType.DMA] * 4 +
            [pltpu.VMEM_SHARED((n_pad, D), jnp.float32)]
        ),
        interpret=interpret,
    )
    def sc_scatter(vals, dst, zero, out, *scr):
        idxs = scr[0:2]
        rows = scr[2:4]
        lsem = scr[4:6]
        asem = scr[6:8]
        acc = scr[8]
        cid = lax.axis_index("c")
        sid = lax.axis_index("s")
        wid = sid * NC + cid
        base = pl.multiple_of(wid * per_w, 8)
        arow = pl.multiple_of(sid * npt, 8)

        pltpu.sync_copy(zero.at[pl.ds(0, cbs)], rows[0])
        for o, sz in pieces:
            pltpu.sync_copy(rows[0].at[pl.ds(0, sz)],
                            acc.at[pl.ds(arow + o, sz)])
        plsc.subcore_barrier()

        def off_of(c):
            return pl.multiple_of(base + c * cbs, 8)

        def l_copies(b, c):
            off = off_of(c)
            return [pltpu.make_async_copy(vals.at[pl.ds(off, cbs)], rows[b],
                                          lsem[b]),
                    pltpu.make_async_copy(dst.at[pl.ds(off, cbs)], idxs[b],
                                          lsem[b])]

        def start_load(b, c):
            for cp in l_copies(b, c):
                cp.start()

        def wait_load(b, c):
            for cp in l_copies(b, c):
                cp.wait()

        def a_copy(b):
            return pltpu.make_async_copy(rows[b], acc.at[idxs[b]], asem[b])

        def start_adds(b):
            a_copy(b).start(add=True)

        def wait_adds(b):
            a_copy(b).wait()

        start_load(0, 0)
        wait_load(0, 0)
        start_adds(0)
        start_load(1, 1)

        def step(b, c):
            wait_load(b, c)
            start_adds(b)
            wait_adds(1 - b)
            start_load(1 - b, c + 1)

        def body(k, carry):
            step(1, 2 * k + 1)
            step(0, 2 * k + 2)
            return carry

        lax.fori_loop(0, (nchs - 3) // 2, body, 0)
        wait_load(1, nchs - 2)
        start_adds(1)
        wait_adds(0)
        start_load(0, nchs - 1)
        wait_load(0, nchs - 1)
        start_adds(0)
        wait_adds(1)
        wait_adds(0)

        plsc.subcore_barrier()
        obase = pl.multiple_of(cid * n_pad + arow, 8)
        for o, sz in pieces:
            pltpu.sync_copy(acc.at[pl.ds(arow + o, sz)],
                            rows[0].at[pl.ds(0, sz)])
            pltpu.sync_copy(rows[0].at[pl.ds(0, sz)],
                            out.at[pl.ds(obase + o, sz)])

    bn = 2000 if n % 2000 == 0 else n
    gn = n // bn
    be = 4000 if e % 4000 == 0 else e
    ge = e // be

    def node_embed_body(x_ref, w_ref, b_ref, wab_ref, nf_ref, a_ref, b2_ref):
        nf = _relu(_dot(x_ref[...], w_ref[...]) + b_ref[...])
        nf_ref[...] = nf
        ab = _dot(nf, wab_ref[...])
        a_ref[...] = ab[:, :D]
        b2_ref[...] = ab[:, D:]

    node_embed = pl.pallas_call(
        node_embed_body,
        grid=(gn,),
        in_specs=[
            pl.BlockSpec((bn, D), lambda i: (i, 0)),
            pl.BlockSpec((D, D), lambda i: (0, 0)),
            pl.BlockSpec((1, D), lambda i: (0, 0)),
            pl.BlockSpec((D, 2 * D), lambda i: (0, 0)),
        ],
        out_specs=[
            pl.BlockSpec((bn, D), lambda i: (i, 0)),
            pl.BlockSpec((bn, D), lambda i: (i, 0)),
            pl.BlockSpec((bn, D), lambda i: (i, 0)),
        ],
        out_shape=[jax.ShapeDtypeStruct((n, D), jnp.float32)] * 3,
        interpret=interpret,
    )

    def edge_embed_body(ea_ref, w_ref, b_ref, ef_ref):
        ef = _relu(_dot(ea_ref[...], w_ref[...]) + b_ref[...])
        ef_ref[...] = jnp.concatenate(
            [ef, jnp.zeros((ef.shape[0], D - DE), jnp.float32)], axis=1)

    edge_embed = pl.pallas_call(
        edge_embed_body,
        grid=(ge,),
        in_specs=[
            pl.BlockSpec((be, 16), lambda i: (i, 0)),
            pl.BlockSpec((16, DE), lambda i: (0, 0)),
            pl.BlockSpec((1, DE), lambda i: (0, 0)),
        ],
        out_specs=pl.BlockSpec((be, D), lambda i: (i, 0)),
        out_shape=jax.ShapeDtypeStruct((e, D), jnp.float32),
        interpret=interpret,
    )

    def edge_mlp_body(sa_ref, sb_ref, ef_ref, wc_ref, b1_ref, w2_ref, b2_ref,
                      out_ref):
        h = _relu(sa_ref[...] + sb_ref[...]
                  + _dot(ef_ref[:, :DE], wc_ref[...]) + b1_ref[...])
        ef2 = _relu(_dot(h, w2_ref[...]) + b2_ref[...])
        out_ref[...] = jnp.concatenate(
            [ef2, jnp.zeros((ef2.shape[0], D - DE), jnp.float32)], axis=1)

    edge_mlp = pl.pallas_call(
        edge_mlp_body,
        grid=(ge,),
        in_specs=[
            pl.BlockSpec((be, D), lambda i: (i, 0)),
            pl.BlockSpec((be, D), lambda i: (i, 0)),
            pl.BlockSpec((be, D), lambda i: (i, 0)),
            pl.BlockSpec((DE, H), lambda i: (0, 0)),
            pl.BlockSpec((1, H), lambda i: (0, 0)),
            pl.BlockSpec((H, DE), lambda i: (0, 0)),
            pl.BlockSpec((1, DE), lambda i: (0, 0)),
        ],
        out_specs=pl.BlockSpec((be, D), lambda i: (i, 0)),
        out_shape=jax.ShapeDtypeStruct((e, D), jnp.float32),
        interpret=interpret,
    )

    def node_update_body(nf_ref, g0_ref, g1_ref, wa_ref, wb_ref, b_ref,
                         wab_ref, nf2_ref, a_ref, b2_ref):
        agg = g0_ref[:, :DE] + g1_ref[:, :DE]
        nf2 = _relu(_dot(nf_ref[...], wa_ref[...]) + _dot(agg, wb_ref[...])
                    + b_ref[...])
        nf2_ref[...] = nf2
        ab = _dot(nf2, wab_ref[...])
        a_ref[...] = ab[:, :D]
        b2_ref[...] = ab[:, D:]

    node_update = pl.pallas_call(
        node_update_body,
        grid=(gn,),
        in_specs=[
            pl.BlockSpec((bn, D), lambda i: (i, 0)),
            pl.BlockSpec((bn, D), lambda i: (i, 0)),
            pl.BlockSpec((bn, D), lambda i: (i, 0)),
            pl.BlockSpec((D, D), lambda i: (0, 0)),
            pl.BlockSpec((DE, D), lambda i: (0, 0)),
            pl.BlockSpec((1, D), lambda i: (0, 0)),
            pl.BlockSpec((D, 2 * D), lambda i: (0, 0)),
        ],
        out_specs=[
            pl.BlockSpec((bn, D), lambda i: (i, 0)),
            pl.BlockSpec((bn, D), lambda i: (i, 0)),
            pl.BlockSpec((bn, D), lambda i: (i, 0)),
        ],
        out_shape=[jax.ShapeDtypeStruct((n, D), jnp.float32)] * 3,
        interpret=interpret,
    )

    def node_final_body(nf_ref, g0_ref, g1_ref, wa_ref, wb_ref, b_ref,
                        wt1_ref, bt1_ref, wt2_ref, bt2_ref, p_ref):
        agg = g0_ref[:, :DE] + g1_ref[:, :DE]
        nf2 = _relu(_dot(nf_ref[...], wa_ref[...]) + _dot(agg, wb_ref[...])
                    + b_ref[...])
        t = _relu(_dot(nf2, wt1_ref[...]) + bt1_ref[...])
        p_ref[...] = _dot(t, wt2_ref[...]) + bt2_ref[...]

    node_final = pl.pallas_call(
        node_final_body,
        grid=(gn,),
        in_specs=[
            pl.BlockSpec((bn, D), lambda i: (i, 0)),
            pl.BlockSpec((bn, D), lambda i: (i, 0)),
            pl.BlockSpec((bn, D), lambda i: (i, 0)),
            pl.BlockSpec((D, D), lambda i: (0, 0)),
            pl.BlockSpec((DE, D), lambda i: (0, 0)),
            pl.BlockSpec((1, D), lambda i: (0, 0)),
            pl.BlockSpec((D, DE), lambda i: (0, 0)),
            pl.BlockSpec((1, DE), lambda i: (0, 0)),
            pl.BlockSpec((DE, 1), lambda i: (0, 0)),
            pl.BlockSpec((1, 1), lambda i: (0, 0)),
        ],
        out_specs=pl.BlockSpec((bn, 1), lambda i: (i, 0)),
        out_shape=jax.ShapeDtypeStruct((n, 1), jnp.float32),
        interpret=interpret,
    )

    def run(x, edge_attr, edge_index, Wn0, bn0, We0, be0, Wm1, bm1, Wm2, bm2,
            Wu, bu, Wt1, bt1, Wt2, bt2):
        src = edge_index[0].astype(jnp.int32)
        dst = edge_index[1].astype(jnp.int32)
        WmA = Wm1[:D]
        WmB = Wm1[D:2 * D]
        WmC = Wm1[2 * D:]
        WAB = jnp.concatenate([WmA, WmB], axis=1)
        WuA = Wu[:D]
        WuB = Wu[D:]
        zero = jnp.zeros((n_pad, D), jnp.float32)
        b1r = bm1.reshape(1, -1)
        b2r = bm2.reshape(1, -1)

        nf, A, B = node_embed(x, Wn0, bn0.reshape(1, -1), WAB)
        ef = edge_embed(edge_attr, We0, be0.reshape(1, -1))

        sa, sb = sc_gather2(A, B, src, dst)
        ef = edge_mlp(sa, sb, ef, WmC, b1r, Wm2, b2r)
        agg = sc_scatter(ef, dst, zero)
        nf, A, B = node_update(nf, agg[:n], agg[n_pad:n_pad + n], WuA, WuB,
                               bu.reshape(1, -1), WAB)
        sa, sb = sc_gather2(A, B, src, dst)
        ef = edge_mlp(sa, sb, ef, WmC, b1r, Wm2, b2r)
        agg = sc_scatter(ef, dst, zero)
        preds = node_final(nf, agg[:n], agg[n_pad:n_pad + n], WuA, WuB,
                           bu.reshape(1, -1),
                           Wt1, bt1.reshape(1, -1), Wt2,
                           bt2.reshape(1, -1))
        return preds.squeeze(-1)

    return run


@functools.cache
def _pipeline():
    return _build(10000, 320000)


def kernel(x, edge_attr, edge_index, Wn0, bn0, We0, be0, Wm1, bm1, Wm2, bm2,
           Wu, bu, Wt1, bt1, Wt2, bt2):
    return _pipeline()(x, edge_attr, edge_index, Wn0, bn0, We0, be0,
                       Wm1, bm1, Wm2, bm2, Wu, bu, Wt1, bt1, Wt2, bt2)
